# baseline (speedup 1.0000x reference)
import jax
import jax.numpy as jnp
from jax.experimental import pallas as pl
from jax.experimental.pallas import tpu as pltpu

K = 3
R = 2
N = 512
KT = 3


def _dot(a, b):
    return jax.lax.dot_general(
        a, b, (((1,), (0,)), ((), ())),
        preferred_element_type=jnp.float32)


def _fused_kernel(x_ref, sup_ref, att_ref, w1_ref, wt_ref, wlf_ref, wrf_ref,
                  w2_ref, g_ref, bta_ref, out_ref,
                  mscr, hscr, hbuf, wtc_scr, wl_scr, wr_scr):
    first = pl.program_id(0) == 0

    ch = wt_ref.shape[1]
    cs = wt_ref.shape[2]
    cs1 = cs + 1
    r1 = R + 1

    @pl.when(first)
    def _():
        m0 = (att_ref[0] != 0).astype(jnp.float32)
        m1 = (att_ref[1] != 0).astype(jnp.float32)
        mscr[0] = m0.astype(jnp.bfloat16)
        mscr[1] = (1.0 - m0).astype(jnp.bfloat16)
        mscr[2] = m1.astype(jnp.bfloat16)
        mscr[3] = (1.0 - m1).astype(jnp.bfloat16)
        for k in range(K):
            mk = (sup_ref[k] != 0).astype(jnp.float32)
            uk = ((m0 + m1 + mk) > 0).astype(jnp.float32)
            mscr[4 + k] = mk.astype(jnp.bfloat16)
            mscr[7 + k] = ((1.0 - mk) * uk).astype(jnp.bfloat16)
        col = jax.lax.broadcasted_iota(jnp.int32, (ch + 1, K * cs1), 1)
        row = jax.lax.broadcasted_iota(jnp.int32, (ch + 1, K * cs1), 0)
        wtc_scr[...] = jnp.where(
            jnp.logical_and((col % cs1) == cs, row == ch), 1.0, 0.0)
        wl_scr[...] = jnp.zeros((K * r1, K * cs1), jnp.float32)
        wr_scr[...] = jnp.zeros((K * r1, K * cs1), jnp.float32)
        for k in range(K):
            wtc_scr[0:ch, cs1 * k:cs1 * k + cs] = wt_ref[k]
            wl_scr[r1 * k:r1 * (k + 1), cs1 * k:cs1 * k + cs] = wlf_ref[k]
            wr_scr[r1 * k:r1 * (k + 1), cs1 * k:cs1 * k + cs] = wrf_ref[k]

    if True:
        T = x_ref.shape[1]
        cin = x_ref.shape[-1]
        t1 = T - KT + 1
        xf = x_ref[0].reshape(T * N, cin)
        a0 = _dot(xf, w1_ref[0])
        a1 = _dot(xf, w1_ref[1])
        a2 = _dot(xf, w1_ref[2])
        y = a0[:t1 * N] + a1[N:(t1 + 1) * N] + a2[2 * N:(t1 + 2) * N]
        c2 = y.shape[-1] // 2
        h = y[:, :c2] * jax.nn.sigmoid(y[:, c2:])
        hscr[...] = jnp.concatenate(
            [h, jnp.ones((t1 * N, 1), jnp.float32)],
            axis=1).reshape(t1, N, c2 + 1)

    m0 = mscr[0]
    nm0 = mscr[1]
    m1 = mscr[2]
    nm1 = mscr[3]

    def _attn_step(tt):
        ha = hscr[tt]
        wxa = _dot(ha, wtc_scr[...])
        eal = jnp.exp(jax.lax.dot_general(
            wxa, wl_scr[...], (((1,), (1,)), ((), ())),
            preferred_element_type=jnp.float32))
        ear = jnp.exp(jax.lax.dot_general(
            wr_scr[...], wxa, (((1,), (1,)), ((), ())),
            preferred_element_type=jnp.float32))
        ealh = eal.astype(jnp.bfloat16)
        earh = ear.astype(jnp.bfloat16)
        wxah = wxa.astype(jnp.bfloat16)
        attn = jnp.zeros((N, cs), dtype=jnp.float32)
        for k in range(K):
            mk = mscr[4 + k]
            wk = mscr[7 + k]
            c = r1 * k
            f = (m0 * ealh[:, c:c + 1]) * earh[c:c + 1, :] + nm0
            f = f * ((m1 * ealh[:, c + 1:c + 2]) * earh[c + 1:c + 2, :] + nm1)
            f = f * ((mk * ealh[:, c + 2:c + 3]) * earh[c + 2:c + 3, :] + wk)
            ew = _dot(f, wxah[:, cs1 * k:cs1 * (k + 1)])
            attn = attn + (1.0 / ew[:, cs:cs + 1]) * ew[:, :cs]
        attn = jnp.where(attn > 0, attn,
                         jnp.exp(jnp.minimum(attn, 0.0)) - 1.0)
        hbuf[tt] = attn

    for i in range(10):
        _attn_step(i)

    if True:
        t1 = hbuf.shape[0]
        t2 = t1 - KT + 1
        hf = hbuf[...].reshape(t1 * N, cs)
        y2 = _dot(hf[:t2 * N], w2_ref[0])
        y2 = y2 + _dot(hf[N:(t2 + 1) * N], w2_ref[1])
        y2 = y2 + _dot(hf[2 * N:(t2 + 2) * N], w2_ref[2])
        co = y2.shape[-1] // 2
        g = (y2[:, :co] * jax.nn.sigmoid(y2[:, co:])).reshape(t2, N, co)
        mu = jnp.mean(g, axis=(1, 2), keepdims=True)
        var = jnp.mean((g - mu) * (g - mu), axis=(1, 2), keepdims=True)
        out_ref[0] = ((g - mu) / jnp.sqrt(var + 1e-6)) * g_ref[0, 0] \
            + bta_ref[0, 0]


def kernel(x, supports, atten_supports, W1, W_transform, W_left, W_right,
           W2, gamma, beta):
    B, T, n, cin = x.shape
    ch2 = W1.shape[-1]
    ch = ch2 // 2
    cs = W_transform.shape[-1]
    cout2 = W2.shape[-1]
    T1 = T - KT + 1
    T2 = T1 - KT + 1

    out = pl.pallas_call(
        _fused_kernel,
        grid=(B,),
        in_specs=[
            pl.BlockSpec((1, T, n, cin), lambda b: (b, 0, 0, 0)),
            pl.BlockSpec((K, n, n), lambda b: (0, 0, 0)),
            pl.BlockSpec((R, n, n), lambda b: (0, 0, 0)),
            pl.BlockSpec((KT, cin, ch2), lambda b: (0, 0, 0)),
            pl.BlockSpec((K, ch, cs), lambda b: (0, 0, 0)),
            pl.BlockSpec((K, R + 1, cs), lambda b: (0, 0, 0)),
            pl.BlockSpec((K, R + 1, cs), lambda b: (0, 0, 0)),
            pl.BlockSpec((KT, cs, cout2), lambda b: (0, 0, 0)),
            pl.BlockSpec((1, 1, n, cout2 // 2), lambda b: (0, 0, 0, 0)),
            pl.BlockSpec((1, 1, n, cout2 // 2), lambda b: (0, 0, 0, 0)),
        ],
        out_specs=pl.BlockSpec(
            (1, T2, n, cout2 // 2),
            lambda b: (b, 0, 0, 0)),
        out_shape=jax.ShapeDtypeStruct((B, T2, n, cout2 // 2), jnp.float32),
        scratch_shapes=[
            pltpu.VMEM((7 + K, n, n), jnp.bfloat16),
            pltpu.VMEM((T1, n, ch + 1), jnp.float32),
            pltpu.VMEM((T1, n, cs), jnp.float32),
            pltpu.VMEM((ch + 1, K * (cs + 1)), jnp.float32),
            pltpu.VMEM((K * (R + 1), K * (cs + 1)), jnp.float32),
            pltpu.VMEM((K * (R + 1), K * (cs + 1)), jnp.float32),
        ],
        compiler_params=pltpu.CompilerParams(
            dimension_semantics=("arbitrary",)),
    )(x, supports, atten_supports, W1, W_transform, W_left, W_right, W2,
      gamma, beta)
    return out


# speedup vs baseline: 1.0280x; 1.0280x over previous
import jax
import jax.numpy as jnp
from jax.experimental import pallas as pl
from jax.experimental.pallas import tpu as pltpu

K = 3
R = 2
N = 512
KT = 3


def _dot(a, b):
    return jax.lax.dot_general(
        a, b, (((1,), (0,)), ((), ())),
        preferred_element_type=jnp.float32)


def _fused_kernel(x_ref, sup_ref, att_ref, w1_ref, wt_ref, wlf_ref, wrf_ref,
                  w2_ref, g_ref, bta_ref, out_ref,
                  mscr, hscr, hbuf, wtc_scr, wl_scr, wr_scr):
    j = pl.program_id(1)
    first = jnp.logical_and(pl.program_id(0) == 0, j == 0)

    ch = wt_ref.shape[1]
    cs = wt_ref.shape[2]
    cs1 = cs + 1
    r1 = R + 1

    @pl.when(first)
    def _():
        m0 = (att_ref[0] != 0).astype(jnp.float32)
        m1 = (att_ref[1] != 0).astype(jnp.float32)
        mscr[0] = m0.astype(jnp.bfloat16)
        mscr[1] = (1.0 - m0).astype(jnp.bfloat16)
        mscr[2] = m1.astype(jnp.bfloat16)
        mscr[3] = (1.0 - m1).astype(jnp.bfloat16)
        for k in range(K):
            mk = (sup_ref[k] != 0).astype(jnp.float32)
            uk = ((m0 + m1 + mk) > 0).astype(jnp.float32)
            mscr[4 + k] = mk.astype(jnp.bfloat16)
            mscr[7 + k] = ((1.0 - mk) * uk).astype(jnp.bfloat16)
        col = jax.lax.broadcasted_iota(jnp.int32, (ch + 1, K * cs1), 1)
        row = jax.lax.broadcasted_iota(jnp.int32, (ch + 1, K * cs1), 0)
        wtc_scr[...] = jnp.where(
            jnp.logical_and((col % cs1) == cs, row == ch), 1.0, 0.0)
        wl_scr[...] = jnp.zeros((K * r1, K * cs1), jnp.float32)
        wr_scr[...] = jnp.zeros((K * r1, K * cs1), jnp.float32)
        for k in range(K):
            wtc_scr[0:ch, cs1 * k:cs1 * k + cs] = wt_ref[k]
            wl_scr[r1 * k:r1 * (k + 1), cs1 * k:cs1 * k + cs] = wlf_ref[k]
            wr_scr[r1 * k:r1 * (k + 1), cs1 * k:cs1 * k + cs] = wrf_ref[k]

    @pl.when(j == 0)
    def _():
        T = x_ref.shape[1]
        cin = x_ref.shape[-1]
        t1 = T - KT + 1
        xf = x_ref[0].reshape(T * N, cin)
        a0 = _dot(xf, w1_ref[0])
        a1 = _dot(xf, w1_ref[1])
        a2 = _dot(xf, w1_ref[2])
        y = a0[:t1 * N] + a1[N:(t1 + 1) * N] + a2[2 * N:(t1 + 2) * N]
        c2 = y.shape[-1] // 2
        h = y[:, :c2] * jax.nn.sigmoid(y[:, c2:])
        hscr[...] = jnp.concatenate(
            [h, jnp.ones((t1 * N, 1), jnp.float32)],
            axis=1).reshape(t1, N, c2 + 1)

    m0 = mscr[0]
    nm0 = mscr[1]
    m1 = mscr[2]
    nm1 = mscr[3]

    def _attn_step(tt):
        ha = hscr[tt]
        wxa = _dot(ha, wtc_scr[...])
        eal = jnp.exp(jax.lax.dot_general(
            wxa, wl_scr[...], (((1,), (1,)), ((), ())),
            preferred_element_type=jnp.float32))
        ear = jnp.exp(jax.lax.dot_general(
            wr_scr[...], wxa, (((1,), (1,)), ((), ())),
            preferred_element_type=jnp.float32))
        ealh = eal.astype(jnp.bfloat16)
        earh = ear.astype(jnp.bfloat16)
        wxah = wxa.astype(jnp.bfloat16)
        attn = jnp.zeros((N, cs), dtype=jnp.float32)
        for k in range(K):
            mk = mscr[4 + k]
            wk = mscr[7 + k]
            c = r1 * k
            f = (m0 * ealh[:, c:c + 1]) * earh[c:c + 1, :] + nm0
            f = f * ((m1 * ealh[:, c + 1:c + 2]) * earh[c + 1:c + 2, :] + nm1)
            f = f * ((mk * ealh[:, c + 2:c + 3]) * earh[c + 2:c + 3, :] + wk)
            ew = _dot(f, wxah[:, cs1 * k:cs1 * (k + 1)])
            attn = attn + (1.0 / ew[:, cs:cs + 1]) * ew[:, :cs]
        attn = jnp.where(attn > 0, attn,
                         jnp.exp(jnp.minimum(attn, 0.0)) - 1.0)
        hbuf[tt] = attn

    for i in range(5):
        _attn_step(5 * j + i)

    @pl.when(j == 1)
    def _():
        t1 = hbuf.shape[0]
        t2 = t1 - KT + 1
        hf = hbuf[...].reshape(t1 * N, cs)
        y2 = _dot(hf[:t2 * N], w2_ref[0])
        y2 = y2 + _dot(hf[N:(t2 + 1) * N], w2_ref[1])
        y2 = y2 + _dot(hf[2 * N:(t2 + 2) * N], w2_ref[2])
        co = y2.shape[-1] // 2
        g = (y2[:, :co] * jax.nn.sigmoid(y2[:, co:])).reshape(t2, N, co)
        mu = jnp.mean(g, axis=(1, 2), keepdims=True)
        var = jnp.mean((g - mu) * (g - mu), axis=(1, 2), keepdims=True)
        out_ref[0] = ((g - mu) / jnp.sqrt(var + 1e-6)) * g_ref[0, 0] \
            + bta_ref[0, 0]


def kernel(x, supports, atten_supports, W1, W_transform, W_left, W_right,
           W2, gamma, beta):
    B, T, n, cin = x.shape
    ch2 = W1.shape[-1]
    ch = ch2 // 2
    cs = W_transform.shape[-1]
    cout2 = W2.shape[-1]
    T1 = T - KT + 1
    T2 = T1 - KT + 1

    out = pl.pallas_call(
        _fused_kernel,
        grid=(B, 2),
        in_specs=[
            pl.BlockSpec((1, T, n, cin), lambda b, t: (b, 0, 0, 0)),
            pl.BlockSpec((K, n, n), lambda b, t: (0, 0, 0)),
            pl.BlockSpec((R, n, n), lambda b, t: (0, 0, 0)),
            pl.BlockSpec((KT, cin, ch2), lambda b, t: (0, 0, 0)),
            pl.BlockSpec((K, ch, cs), lambda b, t: (0, 0, 0)),
            pl.BlockSpec((K, R + 1, cs), lambda b, t: (0, 0, 0)),
            pl.BlockSpec((K, R + 1, cs), lambda b, t: (0, 0, 0)),
            pl.BlockSpec((KT, cs, cout2), lambda b, t: (0, 0, 0)),
            pl.BlockSpec((1, 1, n, cout2 // 2), lambda b, t: (0, 0, 0, 0)),
            pl.BlockSpec((1, 1, n, cout2 // 2), lambda b, t: (0, 0, 0, 0)),
        ],
        out_specs=pl.BlockSpec(
            (1, T2, n, cout2 // 2),
            lambda b, j: (b, 0, 0, 0)),
        out_shape=jax.ShapeDtypeStruct((B, T2, n, cout2 // 2), jnp.float32),
        scratch_shapes=[
            pltpu.VMEM((7 + K, n, n), jnp.bfloat16),
            pltpu.VMEM((T1, n, ch + 1), jnp.float32),
            pltpu.VMEM((T1, n, cs), jnp.float32),
            pltpu.VMEM((ch + 1, K * (cs + 1)), jnp.float32),
            pltpu.VMEM((K * (R + 1), K * (cs + 1)), jnp.float32),
            pltpu.VMEM((K * (R + 1), K * (cs + 1)), jnp.float32),
        ],
        compiler_params=pltpu.CompilerParams(
            dimension_semantics=("arbitrary", "arbitrary")),
    )(x, supports, atten_supports, W1, W_transform, W_left, W_right, W2,
      gamma, beta)
    return out
